# baseline (speedup 1.0000x reference)
import functools
import jax
import jax.numpy as jnp
from jax import lax
from jax.experimental import pallas as pl
from jax.experimental.pallas import tpu as pltpu
from jax.experimental.pallas import tpu_sc as plsc

B = 16384
N_ACTIONS = 1000000
DIM = 32
NC = 2
NS = 16
NW = NC * NS
BPW = B // NW


def _sc_body(ctx_hbm, qry_hbm, act_hbm, tbl_hbm, wb_hbm, out_hbm,
             idx_v, rows_v, ctx_v, qry_v, wb_v, out_v, sem, sem2):
    wid = lax.axis_index("s") * NC + lax.axis_index("c")
    base = wid * BPW

    pltpu.sync_copy(act_hbm.at[pl.ds(base, BPW)], idx_v)
    gather = pltpu.async_copy(tbl_hbm.at[idx_v], rows_v, sem)
    cp_c = pltpu.async_copy(ctx_hbm.at[pl.ds(base * DIM, BPW * DIM)], ctx_v, sem2)
    cp_q = pltpu.async_copy(qry_hbm.at[pl.ds(base * DIM, BPW * DIM)], qry_v, sem2)
    pltpu.sync_copy(wb_hbm, wb_v)

    w0 = wb_v[pl.ds(0, 16)]
    w1 = wb_v[pl.ds(16, 16)]
    w2 = wb_v[pl.ds(32, 16)]
    w3 = wb_v[pl.ds(48, 16)]
    w4 = wb_v[pl.ds(64, 16)]
    w5 = wb_v[pl.ds(80, 16)]
    bias = wb_v[pl.ds(96, 16)][0]
    lanes = lax.iota(jnp.int32, 16)

    cp_c.wait()
    cp_q.wait()
    gather.wait()

    @plsc.parallel_loop(0, BPW // 16, step=1, unroll=2)
    def body(g):
        acc = jnp.zeros((16,), jnp.float32)
        for j in range(16):
            s = g * 16 + j
            c0 = ctx_v[pl.ds(s * DIM, 16)]
            c1 = ctx_v[pl.ds(s * DIM + 16, 16)]
            q0 = qry_v[pl.ds(s * DIM, 16)]
            q1 = qry_v[pl.ds(s * DIM + 16, 16)]
            a0 = rows_v[s, pl.ds(0, 16)]
            a1 = rows_v[s, pl.ds(16, 16)]
            t = c0 * w0 + c1 * w1 + q0 * w2 + q1 * w3 + a0 * w4 + a1 * w5
            acc = jnp.where(lanes == j, jnp.sum(t) + bias, acc)
        out_v[pl.ds(g * 16, 16)] = acc

    pltpu.sync_copy(out_v, out_hbm.at[pl.ds(base, BPW)])


@jax.jit
def _run(ctx_flat, qry_flat, action, action_list, wb):
    mesh = plsc.VectorSubcoreMesh(core_axis_name="c", subcore_axis_name="s",
                                  num_cores=NC, num_subcores=NS)
    f = pl.kernel(
        _sc_body,
        out_type=jax.ShapeDtypeStruct((B,), jnp.float32),
        mesh=mesh,
        scratch_types=[
            pltpu.VMEM((BPW,), jnp.int32),
            pltpu.VMEM((BPW, DIM), jnp.float32),
            pltpu.VMEM((BPW * DIM,), jnp.float32),
            pltpu.VMEM((BPW * DIM,), jnp.float32),
            pltpu.VMEM((112,), jnp.float32),
            pltpu.VMEM((BPW,), jnp.float32),
            pltpu.SemaphoreType.DMA,
            pltpu.SemaphoreType.DMA,
        ],
        compiler_params=pltpu.CompilerParams(needs_layout_passes=False,
                                             use_tc_tiling_on_sc=False),
    )
    return f(ctx_flat, qry_flat, action, action_list, wb)


def kernel(context, query, action, action_list, w, b):
    wb = jnp.concatenate(
        [w, jnp.reshape(b, (1,)), jnp.zeros((15,), jnp.float32)])
    return _run(context.reshape(-1), query.reshape(-1),
                action.astype(jnp.int32), action_list, wb)
